# baseline (speedup 1.0000x reference)
import functools

import jax
import jax.numpy as jnp
from jax import lax
from jax.experimental import pallas as pl
from jax.experimental.pallas import tpu as pltpu
from jax.experimental.pallas import tpu_sc as plsc

N = 10000
E = 320000
D = 128
DH = D // 2
BN_EPS = 1e-5

NC = 2
NS = 16
EPW = E // NS
CHUNK = 128
NCHUNK = 158
EPWP = NCHUNK * CHUNK
NP = 10240
RPS = NP // NS
L = 16


@functools.partial(
    pl.kernel,
    mesh=plsc.VectorSubcoreMesh(core_axis_name="c", subcore_axis_name="s"),
    out_type=jax.ShapeDtypeStruct((NC, NP, DH), jnp.float32),
    scratch_types=[
        pltpu.VMEM((NCHUNK, CHUNK), jnp.int32),
        pltpu.VMEM((NCHUNK, CHUNK), jnp.int32),
        pltpu.VMEM((CHUNK, DH), jnp.float32),
        pltpu.VMEM((CHUNK, DH), jnp.float32),
        pltpu.VMEM_SHARED((NP, DH), jnp.float32),
        pltpu.SemaphoreType.DMA,
        pltpu.SemaphoreType.DMA,
    ],
    compiler_params=pltpu.CompilerParams(use_tc_tiling_on_sc=False),
)
def _agg_kernel(xh_hbm, src_hbm, dst_hbm, zero_hbm, out_hbm,
                idx_v, dst_v, rows0, rows1, acc_sh, sem0, sem1):
    cid = lax.axis_index("c")
    sid = lax.axis_index("s")

    r0 = sid * RPS
    pltpu.sync_copy(zero_hbm.at[pl.ds(r0, RPS)], acc_sh.at[pl.ds(r0, RPS)])

    pltpu.sync_copy(src_hbm.at[cid * NS + sid], idx_v)
    pltpu.sync_copy(dst_hbm.at[sid], dst_v)

    plsc.subcore_barrier()

    pltpu.async_copy(xh_hbm.at[idx_v.at[0]], rows0, sem0)
    pltpu.async_copy(xh_hbm.at[idx_v.at[1]], rows1, sem1)

    def body(i, carry):
        g = 2 * i
        pltpu.make_async_copy(xh_hbm.at[idx_v.at[g]], rows0, sem0).wait()
        pltpu.sync_copy(rows0, acc_sh.at[dst_v.at[g]], add=True)

        @pl.when(g + 2 < NCHUNK)
        def _():
            pltpu.async_copy(xh_hbm.at[idx_v.at[g + 2]], rows0, sem0)

        pltpu.make_async_copy(xh_hbm.at[idx_v.at[g + 1]], rows1, sem1).wait()
        pltpu.sync_copy(rows1, acc_sh.at[dst_v.at[g + 1]], add=True)

        @pl.when(g + 3 < NCHUNK)
        def _():
            pltpu.async_copy(xh_hbm.at[idx_v.at[g + 3]], rows1, sem1)

        return carry

    lax.fori_loop(0, NCHUNK // 2, body, 0)

    plsc.subcore_barrier()

    pltpu.sync_copy(acc_sh.at[pl.ds(r0, RPS)],
                    out_hbm.at[cid, pl.ds(r0, RPS)])


def _mlp_body(x_ref, agg_ref, eps_ref, a_ref,
              W1_ref, b1_ref, g1_ref, be1_ref,
              W2_ref, b2_ref, g2_ref, be2_ref, out_ref):
    agg = jnp.concatenate([agg_ref[0, :N], agg_ref[1, :N]], axis=1)
    h = (1.0 + eps_ref[0]) * x_ref[...] + agg

    h = lax.dot_general(h, W1_ref[...], (((1,), (1,)), ((), ())),
                        preferred_element_type=jnp.float32) + b1_ref[...]
    m = jnp.mean(h, axis=0, keepdims=True)
    v = jnp.mean((h - m) ** 2, axis=0, keepdims=True)
    h = (h - m) * lax.rsqrt(v + BN_EPS)
    h = g1_ref[...] * h + be1_ref[...]
    h = jnp.where(h >= 0.0, h, a_ref[0] * h)

    h = lax.dot_general(h, W2_ref[...], (((1,), (1,)), ((), ())),
                        preferred_element_type=jnp.float32) + b2_ref[...]
    m = jnp.mean(h, axis=0, keepdims=True)
    v = jnp.mean((h - m) ** 2, axis=0, keepdims=True)
    h = (h - m) * lax.rsqrt(v + BN_EPS)
    h = g2_ref[...] * h + be2_ref[...]
    out_ref[...] = jnp.where(h >= 0.0, h, a_ref[1] * h)


def _mlp_call(x, agg, eps, a12, W1, b1, g1, be1, W2, b2, g2, be2):
    return pl.pallas_call(
        _mlp_body,
        out_shape=jax.ShapeDtypeStruct((N, D), jnp.float32),
        in_specs=[
            pl.BlockSpec(memory_space=pltpu.VMEM),
            pl.BlockSpec(memory_space=pltpu.VMEM),
            pl.BlockSpec(memory_space=pltpu.SMEM),
            pl.BlockSpec(memory_space=pltpu.SMEM),
            pl.BlockSpec(memory_space=pltpu.VMEM),
            pl.BlockSpec(memory_space=pltpu.VMEM),
            pl.BlockSpec(memory_space=pltpu.VMEM),
            pl.BlockSpec(memory_space=pltpu.VMEM),
            pl.BlockSpec(memory_space=pltpu.VMEM),
            pl.BlockSpec(memory_space=pltpu.VMEM),
            pl.BlockSpec(memory_space=pltpu.VMEM),
            pl.BlockSpec(memory_space=pltpu.VMEM),
        ],
        out_specs=pl.BlockSpec(memory_space=pltpu.VMEM),
    )(x, agg, eps, a12, W1, b1, g1, be1, W2, b2, g2, be2)


def kernel(x, edge_index, eps, W1, b1, g1, be1, a1, W2, b2, g2, be2, a2):
    pad = EPWP - EPW
    src = edge_index[0].astype(jnp.int32)
    src2 = jnp.pad(jnp.stack([2 * src, 2 * src + 1]).reshape(NC * NS, EPW),
                   ((0, 0), (0, pad)), constant_values=0
                   ).reshape(NC * NS, NCHUNK, CHUNK)
    dst_r = jnp.pad(edge_index[1].astype(jnp.int32).reshape(NS, EPW),
                    ((0, 0), (0, pad)), constant_values=N
                    ).reshape(NS, NCHUNK, CHUNK)
    xh = x.reshape(2 * N, DH)
    zeros = jnp.zeros((NP, DH), jnp.float32)

    agg = _agg_kernel(xh, src2, dst_r, zeros)

    eps1 = jnp.reshape(eps, (1,))
    a12 = jnp.stack([a1, a2])
    row = lambda t: jnp.reshape(t, (1, D))
    return _mlp_call(x, agg, eps1, a12, W1, row(b1), row(g1), row(be1),
                     W2, row(b2), row(g2), row(be2))


# speedup vs baseline: 1.7335x; 1.7335x over previous
import functools

import jax
import jax.numpy as jnp
from jax import lax
from jax.experimental import pallas as pl
from jax.experimental.pallas import tpu as pltpu
from jax.experimental.pallas import tpu_sc as plsc

N = 10000
E = 320000
D = 128
DH = D // 2
BN_EPS = 1e-5

NC = 2
NS = 16
EPW = E // NS
CHUNK = 125
NCHUNK = EPW // CHUNK
NP = 10240
RPS = NP // NS
L = 16


@functools.partial(
    pl.kernel,
    mesh=plsc.VectorSubcoreMesh(core_axis_name="c", subcore_axis_name="s"),
    out_type=jax.ShapeDtypeStruct((NP, D), jnp.float32),
    scratch_types=[
        pltpu.VMEM((NCHUNK, CHUNK), jnp.int32),
        pltpu.VMEM((NCHUNK, CHUNK), jnp.int32),
        pltpu.VMEM((CHUNK, DH), jnp.float32),
        pltpu.VMEM((CHUNK, DH), jnp.float32),
        pltpu.VMEM_SHARED((NP, DH), jnp.float32),
        pltpu.SemaphoreType.DMA,
        pltpu.SemaphoreType.DMA,
    ],
    compiler_params=pltpu.CompilerParams(use_tc_tiling_on_sc=False),
)
def _agg_kernel(xh_hbm, src_hbm, dst_hbm, zero_hbm, out_hbm,
                idx_v, dst_v, rows0, rows1, acc_sh, sem0, sem1):
    cid = lax.axis_index("c")
    sid = lax.axis_index("s")

    r0 = sid * RPS
    pltpu.sync_copy(zero_hbm.at[pl.ds(r0, RPS)], acc_sh.at[pl.ds(r0, RPS)])

    pltpu.sync_copy(src_hbm.at[cid * NS + sid], idx_v)
    pltpu.sync_copy(dst_hbm.at[sid], dst_v)

    plsc.subcore_barrier()

    pltpu.async_copy(xh_hbm.at[idx_v.at[0]], rows0, sem0)
    pltpu.async_copy(xh_hbm.at[idx_v.at[1]], rows1, sem1)

    def body(i, carry):
        g = 2 * i
        pltpu.make_async_copy(xh_hbm.at[idx_v.at[g]], rows0, sem0).wait()
        pltpu.sync_copy(rows0, acc_sh.at[dst_v.at[g]], add=True)

        @pl.when(g + 2 < NCHUNK)
        def _():
            pltpu.async_copy(xh_hbm.at[idx_v.at[g + 2]], rows0, sem0)

        pltpu.make_async_copy(xh_hbm.at[idx_v.at[g + 1]], rows1, sem1).wait()
        pltpu.sync_copy(rows1, acc_sh.at[dst_v.at[g + 1]], add=True)

        @pl.when(g + 3 < NCHUNK)
        def _():
            pltpu.async_copy(xh_hbm.at[idx_v.at[g + 3]], rows1, sem1)

        return carry

    lax.fori_loop(0, NCHUNK // 2, body, 0)

    plsc.subcore_barrier()

    pltpu.sync_copy(acc_sh.at[pl.ds(r0, RPS)],
                    out_hbm.at[pl.ds(r0, RPS), pl.ds(cid * DH, DH)])


def _mlp_body(x_ref, agg_ref, eps_ref, a_ref,
              W1_ref, b1_ref, g1_ref, be1_ref,
              W2_ref, b2_ref, g2_ref, be2_ref, out_ref):
    h = (1.0 + eps_ref[0]) * x_ref[...] + agg_ref[:N]

    h = lax.dot_general(h, W1_ref[...], (((1,), (1,)), ((), ())),
                        preferred_element_type=jnp.float32) + b1_ref[...]
    m = jnp.mean(h, axis=0, keepdims=True)
    v = jnp.mean((h - m) ** 2, axis=0, keepdims=True)
    h = (h - m) * lax.rsqrt(v + BN_EPS)
    h = g1_ref[...] * h + be1_ref[...]
    h = jnp.where(h >= 0.0, h, a_ref[0] * h)

    h = lax.dot_general(h, W2_ref[...], (((1,), (1,)), ((), ())),
                        preferred_element_type=jnp.float32) + b2_ref[...]
    m = jnp.mean(h, axis=0, keepdims=True)
    v = jnp.mean((h - m) ** 2, axis=0, keepdims=True)
    h = (h - m) * lax.rsqrt(v + BN_EPS)
    h = g2_ref[...] * h + be2_ref[...]
    out_ref[...] = jnp.where(h >= 0.0, h, a_ref[1] * h)


def _mlp_call(x, agg, eps, a12, W1, b1, g1, be1, W2, b2, g2, be2):
    return pl.pallas_call(
        _mlp_body,
        out_shape=jax.ShapeDtypeStruct((N, D), jnp.float32),
        in_specs=[
            pl.BlockSpec(memory_space=pltpu.VMEM),
            pl.BlockSpec(memory_space=pltpu.VMEM),
            pl.BlockSpec(memory_space=pltpu.SMEM),
            pl.BlockSpec(memory_space=pltpu.SMEM),
            pl.BlockSpec(memory_space=pltpu.VMEM),
            pl.BlockSpec(memory_space=pltpu.VMEM),
            pl.BlockSpec(memory_space=pltpu.VMEM),
            pl.BlockSpec(memory_space=pltpu.VMEM),
            pl.BlockSpec(memory_space=pltpu.VMEM),
            pl.BlockSpec(memory_space=pltpu.VMEM),
            pl.BlockSpec(memory_space=pltpu.VMEM),
            pl.BlockSpec(memory_space=pltpu.VMEM),
        ],
        out_specs=pl.BlockSpec(memory_space=pltpu.VMEM),
    )(x, agg, eps, a12, W1, b1, g1, be1, W2, b2, g2, be2)


def kernel(x, edge_index, eps, W1, b1, g1, be1, a1, W2, b2, g2, be2, a2):
    src = edge_index[0].astype(jnp.int32)
    src2 = jnp.stack([2 * src, 2 * src + 1]).reshape(NC * NS, NCHUNK, CHUNK)
    dst_r = edge_index[1].astype(jnp.int32).reshape(NS, NCHUNK, CHUNK)
    xh = x.reshape(2 * N, DH)
    zeros = jnp.zeros((NP, DH), jnp.float32)

    agg = _agg_kernel(xh, src2, dst_r, zeros)

    eps1 = jnp.reshape(eps, (1,))
    a12 = jnp.stack([a1, a2])
    row = lambda t: jnp.reshape(t, (1, D))
    return _mlp_call(x, agg, eps1, a12, W1, row(b1), row(g1), row(be1),
                     W2, row(b2), row(g2), row(be2))


# speedup vs baseline: 2.0989x; 1.2108x over previous
import functools

import jax
import jax.numpy as jnp
from jax import lax
from jax.experimental import pallas as pl
from jax.experimental.pallas import tpu as pltpu
from jax.experimental.pallas import tpu_sc as plsc

N = 10000
E = 320000
D = 128
DH = D // 2
BN_EPS = 1e-5

NC = 2
NS = 16
EPW = E // NS
CHUNK = 125
NCHUNK = EPW // CHUNK
NP = 10240
RPS = NP // NS
L = 16


@functools.partial(
    pl.kernel,
    mesh=plsc.VectorSubcoreMesh(core_axis_name="c", subcore_axis_name="s"),
    out_type=jax.ShapeDtypeStruct((NP, D), jnp.float32),
    scratch_types=[
        pltpu.VMEM((NCHUNK, CHUNK), jnp.int32),
        pltpu.VMEM((NCHUNK, CHUNK), jnp.int32),
        pltpu.VMEM((CHUNK, DH), jnp.float32),
        pltpu.VMEM((CHUNK, DH), jnp.float32),
        pltpu.VMEM((CHUNK, DH), jnp.float32),
        pltpu.VMEM((CHUNK, DH), jnp.float32),
        pltpu.VMEM_SHARED((NP, DH), jnp.float32),
        pltpu.SemaphoreType.DMA,
        pltpu.SemaphoreType.DMA,
        pltpu.SemaphoreType.DMA,
        pltpu.SemaphoreType.DMA,
    ],
    compiler_params=pltpu.CompilerParams(use_tc_tiling_on_sc=False),
)
def _agg_kernel(xh_hbm, src_hbm, dst_hbm, zero_hbm, out_hbm,
                idx_v, dst_v, rows0, rows1, rows2, rows3, acc_sh,
                sem0, sem1, sem2, sem3):
    cid = lax.axis_index("c")
    sid = lax.axis_index("s")

    r0 = sid * RPS
    pltpu.sync_copy(zero_hbm.at[pl.ds(r0, RPS)], acc_sh.at[pl.ds(r0, RPS)])

    pltpu.sync_copy(src_hbm.at[cid * NS + sid], idx_v)
    pltpu.sync_copy(dst_hbm.at[sid], dst_v)

    plsc.subcore_barrier()

    bufs = ((rows0, sem0), (rows1, sem1), (rows2, sem2), (rows3, sem3))
    for j, (rb, sb) in enumerate(bufs):
        pltpu.async_copy(xh_hbm.at[idx_v.at[j]], rb, sb)

    def body(i, carry):
        g = 4 * i
        for j, (rb, sb) in enumerate(bufs):
            pltpu.make_async_copy(xh_hbm.at[idx_v.at[g + j]], rb, sb).wait()
            pltpu.sync_copy(rb, acc_sh.at[dst_v.at[g + j]], add=True)

            @pl.when(g + j + 4 < NCHUNK)
            def _(rb=rb, sb=sb):
                pltpu.async_copy(xh_hbm.at[idx_v.at[g + j + 4]], rb, sb)

        return carry

    lax.fori_loop(0, NCHUNK // 4, body, 0)

    plsc.subcore_barrier()

    pltpu.sync_copy(acc_sh.at[pl.ds(r0, RPS)],
                    out_hbm.at[pl.ds(r0, RPS), pl.ds(cid * DH, DH)])


def _mlp_body(x_ref, agg_ref, eps_ref, a_ref,
              W1_ref, b1_ref, g1_ref, be1_ref,
              W2_ref, b2_ref, g2_ref, be2_ref, out_ref):
    h = (1.0 + eps_ref[0]) * x_ref[...] + agg_ref[:N]

    h = lax.dot_general(h, W1_ref[...], (((1,), (1,)), ((), ())),
                        preferred_element_type=jnp.float32) + b1_ref[...]
    m = jnp.mean(h, axis=0, keepdims=True)
    v = jnp.mean((h - m) ** 2, axis=0, keepdims=True)
    h = (h - m) * lax.rsqrt(v + BN_EPS)
    h = g1_ref[...] * h + be1_ref[...]
    h = jnp.where(h >= 0.0, h, a_ref[0] * h)

    h = lax.dot_general(h, W2_ref[...], (((1,), (1,)), ((), ())),
                        preferred_element_type=jnp.float32) + b2_ref[...]
    m = jnp.mean(h, axis=0, keepdims=True)
    v = jnp.mean((h - m) ** 2, axis=0, keepdims=True)
    h = (h - m) * lax.rsqrt(v + BN_EPS)
    h = g2_ref[...] * h + be2_ref[...]
    out_ref[...] = jnp.where(h >= 0.0, h, a_ref[1] * h)


def _mlp_call(x, agg, eps, a12, W1, b1, g1, be1, W2, b2, g2, be2):
    return pl.pallas_call(
        _mlp_body,
        out_shape=jax.ShapeDtypeStruct((N, D), jnp.float32),
        in_specs=[
            pl.BlockSpec(memory_space=pltpu.VMEM),
            pl.BlockSpec(memory_space=pltpu.VMEM),
            pl.BlockSpec(memory_space=pltpu.SMEM),
            pl.BlockSpec(memory_space=pltpu.SMEM),
            pl.BlockSpec(memory_space=pltpu.VMEM),
            pl.BlockSpec(memory_space=pltpu.VMEM),
            pl.BlockSpec(memory_space=pltpu.VMEM),
            pl.BlockSpec(memory_space=pltpu.VMEM),
            pl.BlockSpec(memory_space=pltpu.VMEM),
            pl.BlockSpec(memory_space=pltpu.VMEM),
            pl.BlockSpec(memory_space=pltpu.VMEM),
            pl.BlockSpec(memory_space=pltpu.VMEM),
        ],
        out_specs=pl.BlockSpec(memory_space=pltpu.VMEM),
    )(x, agg, eps, a12, W1, b1, g1, be1, W2, b2, g2, be2)


def kernel(x, edge_index, eps, W1, b1, g1, be1, a1, W2, b2, g2, be2, a2):
    src = edge_index[0].astype(jnp.int32)
    src2 = jnp.stack([2 * src, 2 * src + 1]).reshape(NC * NS, NCHUNK, CHUNK)
    dst_r = edge_index[1].astype(jnp.int32).reshape(NS, NCHUNK, CHUNK)
    xh = x.reshape(2 * N, DH)
    zeros = jnp.zeros((NP, DH), jnp.float32)

    agg = _agg_kernel(xh, src2, dst_r, zeros)

    eps1 = jnp.reshape(eps, (1,))
    a12 = jnp.stack([a1, a2])
    row = lambda t: jnp.reshape(t, (1, D))
    return _mlp_call(x, agg, eps1, a12, W1, row(b1), row(g1), row(be1),
                     W2, row(b2), row(g2), row(be2))
